# Initial kernel scaffold; baseline (speedup 1.0000x reference)
import jax
import jax.numpy as jnp
from jax.experimental import pallas as pl


def kernel(pulse_embeddings, pulse_to_dom_idx, num_doms, W, b):
    raise NotImplementedError("write your pallas kernel here")


# speedup vs baseline: 7.1066x; 7.1066x over previous
import functools

import jax
import jax.numpy as jnp
from jax import lax
from jax.experimental import pallas as pl
from jax.experimental.pallas import tpu as pltpu
from jax.experimental.pallas import tpu_sc as plsc

N_PULSES = 320000
NUM_DOMS = 10000
EMBED_DIM = 128
NLANE = 16
NREG = EMBED_DIM // NLANE

NC, NS = 2, 16
NW = NC * NS
DPW = 8 * -(-NUM_DOMS // (NW * 8))
NDP = DPW * NW
CHUNK = 256
NBLK = N_PULSES // 16
SEARCH_ITERS = 15


def _sc_pool(emb, idx):
  mesh = plsc.VectorSubcoreMesh(core_axis_name="c", subcore_axis_name="s")

  @functools.partial(
      pl.kernel,
      mesh=mesh,
      out_type=[
          jax.ShapeDtypeStruct((NDP * EMBED_DIM,), jnp.float32),
          jax.ShapeDtypeStruct((NDP * EMBED_DIM,), jnp.float32),
      ],
      scratch_types=[
          pltpu.VMEM((CHUNK * EMBED_DIM,), jnp.float32),
          pltpu.VMEM((CHUNK,), jnp.int32),
          pltpu.VMEM((DPW * EMBED_DIM,), jnp.float32),
          pltpu.VMEM((DPW * EMBED_DIM,), jnp.float32),
          pltpu.VMEM((16,), jnp.int32),
      ],
  )
  def pool_kernel(emb_hbm, idx_hbm, mean_hbm, max_hbm,
                  rowbuf, idxbuf_v, meanbuf, maxbuf, sbuf):
    wid = lax.axis_index("s") * NC + lax.axis_index("c")
    d0 = wid * DPW
    d1 = jnp.minimum(d0 + DPW, NUM_DOMS)

    zeros = jnp.zeros((NLANE,), jnp.float32)

    def zero_row(r, carry):
      for j in range(NREG):
        meanbuf[pl.ds(r * EMBED_DIM + j * NLANE, NLANE)] = zeros
        maxbuf[pl.ds(r * EMBED_DIM + j * NLANE, NLANE)] = zeros
      return carry
    lax.fori_loop(0, DPW, zero_row, 0)

    def lower_bound(target):
      def it(_, c):
        lo, hi = c
        run = lo < hi
        mid = jnp.where(run, (lo + hi) // 2, 0)
        pltpu.sync_copy(idx_hbm.at[pl.ds(mid * 16, 16)], sbuf)
        probe = sbuf[pl.ds(0, NLANE)]
        ge = probe[0] >= target
        nlo = jnp.where(ge, lo, mid + 1)
        nhi = jnp.where(ge, mid, hi)
        return (jnp.where(run, nlo, lo), jnp.where(run, nhi, hi))
      lo, _ = lax.fori_loop(0, SEARCH_ITERS, it,
                            (jnp.int32(0), jnp.int32(NBLK)))
      return lo

    f0 = lower_bound(d0)
    f1 = lower_bound(d1)
    p0 = jnp.maximum(f0 - 1, 0) * 16
    p1 = f1 * 16
    c0 = p0 // CHUNK
    nch = (p1 + CHUNK - 1) // CHUNK - c0

    def flush(cur, cnt, s, m):
      @pl.when(jnp.logical_and(cur >= d0, cur < d1))
      def _():
        off = (cur - d0) * EMBED_DIM
        cnt_v = jnp.broadcast_to(cnt, (NLANE,))
        for j in range(NREG):
          meanbuf[pl.ds(off + j * NLANE, NLANE)] = s[j] / cnt_v
          maxbuf[pl.ds(off + j * NLANE, NLANE)] = m[j]

    def chunk_body(g, carry):
      base = (c0 + g) * CHUNK
      pltpu.sync_copy(emb_hbm.at[pl.ds(base * EMBED_DIM, CHUNK * EMBED_DIM)],
                      rowbuf)
      pltpu.sync_copy(idx_hbm.at[pl.ds(base, CHUNK)], idxbuf_v)

      def group_body(gq, c):
        iv = idxbuf_v[pl.ds(gq * NLANE, NLANE)]
        for k in range(NLANE):
          cur, cnt, s, m = c
          d = iv[k]
          roff = (gq * NLANE + k) * EMBED_DIM
          row = tuple(rowbuf[pl.ds(roff + j * NLANE, NLANE)]
                      for j in range(NREG))
          change = d != cur

          @pl.when(change)
          def _(cur=cur, cnt=cnt, s=s, m=m):
            flush(cur, cnt, s, m)

          new_s = tuple(jnp.where(change, row[j], s[j] + row[j])
                        for j in range(NREG))
          new_m = tuple(jnp.where(change, row[j], jnp.maximum(m[j], row[j]))
                        for j in range(NREG))
          c = (d, jnp.where(change, jnp.float32(1.0), cnt + 1.0),
               new_s, new_m)
        return c

      return lax.fori_loop(0, CHUNK // NLANE, group_body, carry)

    init = (jnp.int32(-1), jnp.float32(0.0),
            tuple(zeros for _ in range(NREG)),
            tuple(zeros for _ in range(NREG)))
    cur, cnt, s, m = lax.fori_loop(0, nch, chunk_body, init)
    flush(cur, cnt, s, m)

    pltpu.sync_copy(meanbuf, mean_hbm.at[pl.ds(d0 * EMBED_DIM, DPW * EMBED_DIM)])
    pltpu.sync_copy(maxbuf, max_hbm.at[pl.ds(d0 * EMBED_DIM, DPW * EMBED_DIM)])

  return pool_kernel(emb, idx)


def _tc_project(mean_p, max_p, w1t, w2t, b2d):
  RB = NDP // 4

  def mm(mean_ref, max_ref, w1_ref, w2_ref, b_ref, o_ref):
    o_ref[...] = (
        jnp.dot(mean_ref[...], w1_ref[...], preferred_element_type=jnp.float32)
        + jnp.dot(max_ref[...], w2_ref[...], preferred_element_type=jnp.float32)
        + b_ref[...])

  return pl.pallas_call(
      mm,
      grid=(NDP // RB,),
      in_specs=[
          pl.BlockSpec((RB, EMBED_DIM), lambda i: (i, 0)),
          pl.BlockSpec((RB, EMBED_DIM), lambda i: (i, 0)),
          pl.BlockSpec((EMBED_DIM, EMBED_DIM), lambda i: (0, 0)),
          pl.BlockSpec((EMBED_DIM, EMBED_DIM), lambda i: (0, 0)),
          pl.BlockSpec((1, EMBED_DIM), lambda i: (0, 0)),
      ],
      out_specs=pl.BlockSpec((RB, EMBED_DIM), lambda i: (i, 0)),
      out_shape=jax.ShapeDtypeStruct((NDP, EMBED_DIM), jnp.float32),
  )(mean_p, max_p, w1t, w2t, b2d)


def kernel(pulse_embeddings, pulse_to_dom_idx, num_doms, W, b):
  idx = pulse_to_dom_idx.astype(jnp.int32)
  mean_p, max_p = _sc_pool(pulse_embeddings.reshape(-1), idx)
  mean_p = mean_p.reshape(NDP, EMBED_DIM)
  max_p = max_p.reshape(NDP, EMBED_DIM)
  w1t = W[:, :EMBED_DIM].T
  w2t = W[:, EMBED_DIM:].T
  b2d = b.reshape(1, EMBED_DIM)
  out = _tc_project(mean_p, max_p, w1t, w2t, b2d)
  return out[:NUM_DOMS]


# speedup vs baseline: 10.5248x; 1.4810x over previous
import functools

import jax
import jax.numpy as jnp
from jax import lax
from jax.experimental import pallas as pl
from jax.experimental.pallas import tpu as pltpu
from jax.experimental.pallas import tpu_sc as plsc

N_PULSES = 320000
NUM_DOMS = 10000
EMBED_DIM = 128
NLANE = 16
NREG = EMBED_DIM // NLANE

NC, NS = 2, 16
NW = NC * NS
DPW = 8 * -(-NUM_DOMS // (NW * 8))
NDP = DPW * NW
CHUNK = 128
NBLK = N_PULSES // 16
SEARCH_ITERS = 15


def _sc_pool(emb, idx):
  mesh = plsc.VectorSubcoreMesh(core_axis_name="c", subcore_axis_name="s")

  @functools.partial(
      pl.kernel,
      mesh=mesh,
      out_type=[
          jax.ShapeDtypeStruct((NDP * EMBED_DIM,), jnp.float32),
          jax.ShapeDtypeStruct((NDP * EMBED_DIM,), jnp.float32),
      ],
      scratch_types=[
          pltpu.VMEM((CHUNK * EMBED_DIM,), jnp.float32),
          pltpu.VMEM((CHUNK * EMBED_DIM,), jnp.float32),
          pltpu.VMEM((CHUNK,), jnp.int32),
          pltpu.VMEM((CHUNK,), jnp.int32),
          pltpu.VMEM((DPW * EMBED_DIM,), jnp.float32),
          pltpu.VMEM((DPW * EMBED_DIM,), jnp.float32),
          pltpu.VMEM((16,), jnp.int32),
          pltpu.SemaphoreType.DMA,
          pltpu.SemaphoreType.DMA,
      ],
  )
  def pool_kernel(emb_hbm, idx_hbm, mean_hbm, max_hbm,
                  rowbuf_a, rowbuf_b, idxbuf_a, idxbuf_b,
                  meanbuf, maxbuf, sbuf, sem_a, sem_b):
    wid = lax.axis_index("s") * NC + lax.axis_index("c")
    d0 = wid * DPW
    d1 = jnp.minimum(d0 + DPW, NUM_DOMS)

    zeros = jnp.zeros((NLANE,), jnp.float32)

    def zero_row(r, carry):
      for j in range(NREG):
        meanbuf[pl.ds(r * EMBED_DIM + j * NLANE, NLANE)] = zeros
        maxbuf[pl.ds(r * EMBED_DIM + j * NLANE, NLANE)] = zeros
      return carry
    lax.fori_loop(0, DPW, zero_row, 0)

    def lower_bound(target):
      def it(_, c):
        lo, hi = c
        run = lo < hi
        mid = jnp.where(run, (lo + hi) // 2, 0)
        pltpu.sync_copy(idx_hbm.at[pl.ds(mid * 16, 16)], sbuf)
        probe = sbuf[pl.ds(0, NLANE)]
        ge = probe[0] >= target
        nlo = jnp.where(ge, lo, mid + 1)
        nhi = jnp.where(ge, mid, hi)
        return (jnp.where(run, nlo, lo), jnp.where(run, nhi, hi))
      lo, _ = lax.fori_loop(0, SEARCH_ITERS, it,
                            (jnp.int32(0), jnp.int32(NBLK)))
      return lo

    f0 = lower_bound(d0)
    f1 = lower_bound(d1)
    p0 = jnp.maximum(f0 - 1, 0) * 16
    p1 = f1 * 16
    c0 = p0 // CHUNK
    nch = (p1 + CHUNK - 1) // CHUNK - c0

    def flush(cur, cnt, s, m):
      @pl.when(jnp.logical_and(cur >= d0, cur < d1))
      def _():
        off = (cur - d0) * EMBED_DIM
        cnt_v = jnp.broadcast_to(cnt, (NLANE,))
        for j in range(NREG):
          meanbuf[pl.ds(off + j * NLANE, NLANE)] = s[j] / cnt_v
          maxbuf[pl.ds(off + j * NLANE, NLANE)] = m[j]

    NB2 = CHUNK * EMBED_DIM

    def issue(g, rb, ib, sem):
      @pl.when(g < nch)
      def _():
        base = (c0 + g) * CHUNK
        pltpu.make_async_copy(
            emb_hbm.at[pl.ds(base * EMBED_DIM, NB2)], rb, sem).start()
        pltpu.make_async_copy(idx_hbm.at[pl.ds(base, CHUNK)], ib, sem).start()

    def wait(g, rb, ib, sem):
      @pl.when(g < nch)
      def _():
        pltpu.make_async_copy(emb_hbm.at[pl.ds(0, NB2)], rb, sem).wait()
        pltpu.make_async_copy(idx_hbm.at[pl.ds(0, CHUNK)], ib, sem).wait()

    def process(rb, ib, valid, carry):
      def group_body(gq, c):
        iv = ib[pl.ds(gq * NLANE, NLANE)]
        for k in range(NLANE):
          cur, cnt, s, m = c
          d = iv[k]
          roff = (gq * NLANE + k) * EMBED_DIM
          row = tuple(rb[pl.ds(roff + j * NLANE, NLANE)]
                      for j in range(NREG))
          change = jnp.logical_and(d != cur, valid)

          @pl.when(change)
          def _(cur=cur, cnt=cnt, s=s, m=m):
            flush(cur, cnt, s, m)

          new_s = tuple(jnp.where(change, row[j], s[j] + row[j])
                        for j in range(NREG))
          new_m = tuple(jnp.where(change, row[j], jnp.maximum(m[j], row[j]))
                        for j in range(NREG))
          c = (jnp.where(valid, d, cur),
               jnp.where(change, jnp.float32(1.0), cnt + 1.0),
               new_s, new_m)
        return c

      return lax.fori_loop(0, CHUNK // NLANE, group_body, carry)

    def merge(valid, new, old):
      return jax.tree.map(lambda a, b: jnp.where(valid, a, b), new, old)

    init = (jnp.int32(-1), jnp.float32(0.0),
            tuple(zeros for _ in range(NREG)),
            tuple(zeros for _ in range(NREG)))

    issue(jnp.int32(0), rowbuf_a, idxbuf_a, sem_a)

    def pair_body(h, carry):
      g0 = 2 * h
      g1 = g0 + 1
      issue(g1, rowbuf_b, idxbuf_b, sem_b)
      wait(g0, rowbuf_a, idxbuf_a, sem_a)
      carry = process(rowbuf_a, idxbuf_a, jnp.bool_(True), carry)
      issue(g0 + 2, rowbuf_a, idxbuf_a, sem_a)
      wait(g1, rowbuf_b, idxbuf_b, sem_b)
      v1 = g1 < nch
      out = process(rowbuf_b, idxbuf_b, v1, carry)
      return merge(v1, out, carry)

    cur, cnt, s, m = lax.fori_loop(0, (nch + 1) // 2, pair_body, init)
    flush(cur, cnt, s, m)

    pltpu.sync_copy(meanbuf, mean_hbm.at[pl.ds(d0 * EMBED_DIM, DPW * EMBED_DIM)])
    pltpu.sync_copy(maxbuf, max_hbm.at[pl.ds(d0 * EMBED_DIM, DPW * EMBED_DIM)])

  return pool_kernel(emb, idx)


def _tc_project(mean_p, max_p, w1t, w2t, b2d):
  RB = NDP // 4

  def mm(mean_ref, max_ref, w1_ref, w2_ref, b_ref, o_ref):
    o_ref[...] = (
        jnp.dot(mean_ref[...], w1_ref[...], preferred_element_type=jnp.float32)
        + jnp.dot(max_ref[...], w2_ref[...], preferred_element_type=jnp.float32)
        + b_ref[...])

  return pl.pallas_call(
      mm,
      grid=(NDP // RB,),
      in_specs=[
          pl.BlockSpec((RB, EMBED_DIM), lambda i: (i, 0)),
          pl.BlockSpec((RB, EMBED_DIM), lambda i: (i, 0)),
          pl.BlockSpec((EMBED_DIM, EMBED_DIM), lambda i: (0, 0)),
          pl.BlockSpec((EMBED_DIM, EMBED_DIM), lambda i: (0, 0)),
          pl.BlockSpec((1, EMBED_DIM), lambda i: (0, 0)),
      ],
      out_specs=pl.BlockSpec((RB, EMBED_DIM), lambda i: (i, 0)),
      out_shape=jax.ShapeDtypeStruct((NDP, EMBED_DIM), jnp.float32),
  )(mean_p, max_p, w1t, w2t, b2d)


def kernel(pulse_embeddings, pulse_to_dom_idx, num_doms, W, b):
  idx = pulse_to_dom_idx.astype(jnp.int32)
  mean_p, max_p = _sc_pool(pulse_embeddings.reshape(-1), idx)
  mean_p = mean_p.reshape(NDP, EMBED_DIM)
  max_p = max_p.reshape(NDP, EMBED_DIM)
  w1t = W[:, :EMBED_DIM].T
  w2t = W[:, EMBED_DIM:].T
  b2d = b.reshape(1, EMBED_DIM)
  out = _tc_project(mean_p, max_p, w1t, w2t, b2d)
  return out[:NUM_DOMS]


# speedup vs baseline: 12.6203x; 1.1991x over previous
import functools

import jax
import jax.numpy as jnp
from jax import lax
from jax.experimental import pallas as pl
from jax.experimental.pallas import tpu as pltpu
from jax.experimental.pallas import tpu_sc as plsc

N_PULSES = 320000
NUM_DOMS = 10000
EMBED_DIM = 128
NLANE = 16
NREG = EMBED_DIM // NLANE

NC, NS = 2, 16
NW = NC * NS
DPW = 8 * -(-NUM_DOMS // (NW * 8))
NDP = DPW * NW
CHUNK = 128
NBLK = N_PULSES // 16
SEARCH_ITERS = 15


def _sc_pool(emb, idx):
  mesh = plsc.VectorSubcoreMesh(core_axis_name="c", subcore_axis_name="s")

  @functools.partial(
      pl.kernel,
      mesh=mesh,
      out_type=[
          jax.ShapeDtypeStruct((NDP * EMBED_DIM,), jnp.float32),
          jax.ShapeDtypeStruct((NDP * EMBED_DIM,), jnp.float32),
      ],
      scratch_types=[
          pltpu.VMEM((CHUNK * EMBED_DIM,), jnp.float32),
          pltpu.VMEM((CHUNK * EMBED_DIM,), jnp.float32),
          pltpu.VMEM((CHUNK,), jnp.int32),
          pltpu.VMEM((CHUNK,), jnp.int32),
          pltpu.VMEM((DPW * EMBED_DIM,), jnp.float32),
          pltpu.VMEM((DPW * EMBED_DIM,), jnp.float32),
          pltpu.VMEM((16,), jnp.int32),
          pltpu.VMEM((2 * EMBED_DIM,), jnp.float32),
          pltpu.SemaphoreType.DMA,
          pltpu.SemaphoreType.DMA,
      ],
  )
  def pool_kernel(emb_hbm, idx_hbm, mean_hbm, max_hbm,
                  rowbuf_a, rowbuf_b, idxbuf_a, idxbuf_b,
                  meanbuf, maxbuf, sbuf, accbuf, sem_a, sem_b):
    wid = lax.axis_index("s") * NC + lax.axis_index("c")
    d0 = wid * DPW
    d1 = jnp.minimum(d0 + DPW, NUM_DOMS)

    zeros = jnp.zeros((NLANE,), jnp.float32)

    def zero_row(r, carry):
      for j in range(NREG):
        meanbuf[pl.ds(r * EMBED_DIM + j * NLANE, NLANE)] = zeros
        maxbuf[pl.ds(r * EMBED_DIM + j * NLANE, NLANE)] = zeros
      return carry
    lax.fori_loop(0, DPW, zero_row, 0)

    def lower_bound(target):
      def it(_, c):
        lo, hi = c
        run = lo < hi
        mid = jnp.where(run, (lo + hi) // 2, 0)
        pltpu.sync_copy(idx_hbm.at[pl.ds(mid * 16, 16)], sbuf)
        probe = sbuf[pl.ds(0, NLANE)]
        ge = probe[0] >= target
        nlo = jnp.where(ge, lo, mid + 1)
        nhi = jnp.where(ge, mid, hi)
        return (jnp.where(run, nlo, lo), jnp.where(run, nhi, hi))
      lo, _ = lax.fori_loop(0, SEARCH_ITERS, it,
                            (jnp.int32(0), jnp.int32(NBLK)))
      return lo

    f0 = lower_bound(d0)
    f1 = lower_bound(d1)
    p0 = jnp.maximum(f0 - 1, 0) * 16
    p1 = f1 * 16
    c0 = p0 // CHUNK
    nch = (p1 + CHUNK - 1) // CHUNK - c0

    def flush(pred, cur, cnt, s, m):
      def fbody(i, z):
        off = (cur - d0) * EMBED_DIM
        cnt_v = jnp.broadcast_to(cnt, (NLANE,))
        for j in range(NREG):
          meanbuf[pl.ds(off + j * NLANE, NLANE)] = s[j] / cnt_v
          maxbuf[pl.ds(off + j * NLANE, NLANE)] = m[j]
        return z
      lax.fori_loop(0, pred.astype(jnp.int32), fbody, jnp.int32(0))

    NB2 = CHUNK * EMBED_DIM

    def issue(g, rb, ib, sem):
      @pl.when(g < nch)
      def _():
        base = (c0 + g) * CHUNK
        pltpu.make_async_copy(
            emb_hbm.at[pl.ds(base * EMBED_DIM, NB2)], rb, sem).start()
        pltpu.make_async_copy(idx_hbm.at[pl.ds(base, CHUNK)], ib, sem).start()

    def wait(g, rb, ib, sem):
      @pl.when(g < nch)
      def _():
        pltpu.make_async_copy(emb_hbm.at[pl.ds(0, NB2)], rb, sem).wait()
        pltpu.make_async_copy(idx_hbm.at[pl.ds(0, CHUNK)], ib, sem).wait()

    def load_acc():
      sv = tuple(accbuf[pl.ds(j * NLANE, NLANE)] for j in range(NREG))
      mv = tuple(accbuf[pl.ds(EMBED_DIM + j * NLANE, NLANE)]
                 for j in range(NREG))
      return sv, mv

    def store_acc(s, m):
      for j in range(NREG):
        accbuf[pl.ds(j * NLANE, NLANE)] = s[j]
        accbuf[pl.ds(EMBED_DIM + j * NLANE, NLANE)] = m[j]

    def process(rb, ib, cur0, cnt0):
      def group_body(gq, c):
        cur, cnt = c
        iv = ib[pl.ds(gq * NLANE, NLANE)]
        uniform = jnp.logical_and(iv[0] == cur, iv[NLANE - 1] == cur)

        def fast():
          s, m = load_acc()
          for k in range(NLANE):
            roff = (gq * NLANE + k) * EMBED_DIM
            row = tuple(rb[pl.ds(roff + j * NLANE, NLANE)]
                        for j in range(NREG))
            s = tuple(s[j] + row[j] for j in range(NREG))
            m = tuple(jnp.maximum(m[j], row[j]) for j in range(NREG))
          store_acc(s, m)
          return (cur, cnt + jnp.float32(NLANE))

        def slow():
          s, m = load_acc()
          ccur, ccnt = cur, cnt
          for k in range(NLANE):
            d = iv[k]
            roff = (gq * NLANE + k) * EMBED_DIM
            row = tuple(rb[pl.ds(roff + j * NLANE, NLANE)]
                        for j in range(NREG))
            change = d != ccur
            pred = jnp.logical_and(change,
                                   jnp.logical_and(ccur >= d0, ccur < d1))
            flush(pred, ccur, ccnt, s, m)
            s = tuple(jnp.where(change, row[j], s[j] + row[j])
                      for j in range(NREG))
            m = tuple(jnp.where(change, row[j], jnp.maximum(m[j], row[j]))
                      for j in range(NREG))
            ccnt = jnp.where(change, jnp.float32(1.0), ccnt + 1.0)
            ccur = d
          store_acc(s, m)
          return (ccur, ccnt)

        return lax.cond(uniform, fast, slow)

      return lax.fori_loop(0, CHUNK // NLANE, group_body, (cur0, cnt0))

    issue(jnp.int32(0), rowbuf_a, idxbuf_a, sem_a)

    def pair_body(h, c):
      g0 = 2 * h
      g1 = g0 + 1
      issue(g1, rowbuf_b, idxbuf_b, sem_b)
      wait(g0, rowbuf_a, idxbuf_a, sem_a)
      cur, cnt = process(rowbuf_a, idxbuf_a, c[0], c[1])
      issue(g0 + 2, rowbuf_a, idxbuf_a, sem_a)
      wait(g1, rowbuf_b, idxbuf_b, sem_b)
      return lax.cond(g1 < nch,
                      lambda: process(rowbuf_b, idxbuf_b, cur, cnt),
                      lambda: (cur, cnt))

    cur, cnt = lax.fori_loop(0, (nch + 1) // 2, pair_body,
                             (jnp.int32(-1), jnp.float32(0.0)))
    s_f, m_f = load_acc()
    flush(jnp.logical_and(cur >= d0, cur < d1), cur, cnt, s_f, m_f)

    pltpu.sync_copy(meanbuf, mean_hbm.at[pl.ds(d0 * EMBED_DIM, DPW * EMBED_DIM)])
    pltpu.sync_copy(maxbuf, max_hbm.at[pl.ds(d0 * EMBED_DIM, DPW * EMBED_DIM)])

  return pool_kernel(emb, idx)


def _tc_project(mean_p, max_p, w1t, w2t, b2d):
  RB = NDP // 4

  def mm(mean_ref, max_ref, w1_ref, w2_ref, b_ref, o_ref):
    o_ref[...] = (
        jnp.dot(mean_ref[...], w1_ref[...], preferred_element_type=jnp.float32)
        + jnp.dot(max_ref[...], w2_ref[...], preferred_element_type=jnp.float32)
        + b_ref[...])

  return pl.pallas_call(
      mm,
      grid=(NDP // RB,),
      in_specs=[
          pl.BlockSpec((RB, EMBED_DIM), lambda i: (i, 0)),
          pl.BlockSpec((RB, EMBED_DIM), lambda i: (i, 0)),
          pl.BlockSpec((EMBED_DIM, EMBED_DIM), lambda i: (0, 0)),
          pl.BlockSpec((EMBED_DIM, EMBED_DIM), lambda i: (0, 0)),
          pl.BlockSpec((1, EMBED_DIM), lambda i: (0, 0)),
      ],
      out_specs=pl.BlockSpec((RB, EMBED_DIM), lambda i: (i, 0)),
      out_shape=jax.ShapeDtypeStruct((NDP, EMBED_DIM), jnp.float32),
  )(mean_p, max_p, w1t, w2t, b2d)


def kernel(pulse_embeddings, pulse_to_dom_idx, num_doms, W, b):
  idx = pulse_to_dom_idx.astype(jnp.int32)
  mean_p, max_p = _sc_pool(pulse_embeddings.reshape(-1), idx)
  mean_p = mean_p.reshape(NDP, EMBED_DIM)
  max_p = max_p.reshape(NDP, EMBED_DIM)
  w1t = W[:, :EMBED_DIM].T
  w2t = W[:, EMBED_DIM:].T
  b2d = b.reshape(1, EMBED_DIM)
  out = _tc_project(mean_p, max_p, w1t, w2t, b2d)
  return out[:NUM_DOMS]
